# baseline (speedup 1.0000x reference)
import functools

import jax
import jax.numpy as jnp
from jax import lax
from jax.experimental import pallas as pl
from jax.experimental.pallas import tpu as pltpu
from jax.experimental.pallas import tpu_sc as plsc

_NC = 2
_NS = 16
_NW = _NC * _NS
_CHUNK = 128
_NBUF = 4
_X0_FRAC = 0.35


def _pad_to(n, m):
    return (n + m - 1) // m * m


def _sc_agg_body(x0, y1, npad, feat, src_hbm, dst_hbm, g_hbm, zeros_hbm,
                 out_hbm, sidx_v, didx_v, rows_v, acc_sh, *sems):
    cid = lax.axis_index("c")
    sid = lax.axis_index("s")
    rows_per_tile = npad // _NS

    pltpu.sync_copy(zeros_hbm, acc_sh.at[pl.ds(sid * rows_per_tile, rows_per_tile), :])
    plsc.subcore_barrier()

    def run(nch, base):
        pltpu.sync_copy(src_hbm.at[pl.ds(base, nch), :], sidx_v.at[pl.ds(0, nch), :])
        pltpu.sync_copy(dst_hbm.at[pl.ds(base, nch), :], didx_v.at[pl.ds(0, nch), :])

        def group(gi, carry):
            descs = []
            for b in range(_NBUF):
                j = gi * _NBUF + b
                descs.append(pltpu.async_copy(g_hbm.at[sidx_v.at[j]], rows_v.at[b], sems[b]))
            for b in range(_NBUF):
                j = gi * _NBUF + b
                descs[b].wait()
                pltpu.sync_copy(rows_v.at[b], acc_sh.at[didx_v.at[j]], add=True)
            return carry

        lax.fori_loop(0, nch // _NBUF, group, 0)

    pl.when(cid == 0)(lambda: run(x0, sid * x0))
    pl.when(cid == 1)(lambda: run(y1, _NS * x0 + sid * y1))
    plsc.subcore_barrier()

    pltpu.sync_copy(
        acc_sh.at[pl.ds(sid * rows_per_tile, rows_per_tile), :],
        out_hbm.at[cid, pl.ds(sid * rows_per_tile, rows_per_tile), :])


def _make_sc_agg(epad, npad, feat):
    per_pair = epad // (_NS * _CHUNK)
    x0 = round(per_pair * _X0_FRAC / 8) * 8
    y1 = per_pair - x0
    mesh = plsc.VectorSubcoreMesh(core_axis_name="c", subcore_axis_name="s")
    body = functools.partial(_sc_agg_body, x0, y1, npad, feat)
    mx = max(x0, y1)
    return pl.kernel(
        body,
        out_type=jax.ShapeDtypeStruct((_NC, npad, feat), jnp.float32),
        mesh=mesh,
        compiler_params=pltpu.CompilerParams(use_tc_tiling_on_sc=False),
        scratch_types=[
            pltpu.VMEM((mx, _CHUNK), jnp.int32),
            pltpu.VMEM((mx, _CHUNK), jnp.int32),
            pltpu.VMEM((_NBUF, _CHUNK, feat), jnp.float32),
            pltpu.VMEM_SHARED((npad, feat), jnp.float32),
        ] + [pltpu.SemaphoreType.DMA] * _NBUF,
    )


def _sc_deg_body(nchunks, npad, dst_hbm, ones_hbm, zeros_hbm, out_hbm,
                 didx_v, ones_v, acc_sh, sem):
    cid = lax.axis_index("c")
    sid = lax.axis_index("s")
    wid = sid * _NC + cid
    rows_per_tile = npad // _NS

    pltpu.sync_copy(dst_hbm.at[pl.ds(wid * nchunks, nchunks), :], didx_v)
    pltpu.sync_copy(ones_hbm, ones_v)
    pltpu.sync_copy(zeros_hbm, acc_sh.at[pl.ds(sid * rows_per_tile, rows_per_tile), :])
    plsc.subcore_barrier()

    def body(j, carry):
        pltpu.sync_copy(ones_v, acc_sh.at[didx_v.at[j]], add=True)
        return carry

    lax.fori_loop(0, nchunks, body, 0)
    plsc.subcore_barrier()

    pltpu.sync_copy(
        acc_sh.at[pl.ds(sid * rows_per_tile, rows_per_tile), :],
        out_hbm.at[cid, pl.ds(sid * rows_per_tile, rows_per_tile), :])


def _make_sc_deg(epad, npad):
    nchunks = epad // (_NW * _CHUNK)
    mesh = plsc.VectorSubcoreMesh(core_axis_name="c", subcore_axis_name="s")
    body = functools.partial(_sc_deg_body, nchunks, npad)
    return pl.kernel(
        body,
        out_type=jax.ShapeDtypeStruct((_NC, npad, 1), jnp.float32),
        mesh=mesh,
        compiler_params=pltpu.CompilerParams(use_tc_tiling_on_sc=False),
        scratch_types=[
            pltpu.VMEM((nchunks, _CHUNK), jnp.int32),
            pltpu.VMEM((_CHUNK, 1), jnp.float32),
            pltpu.VMEM_SHARED((npad, 1), jnp.float32),
            pltpu.SemaphoreType.DMA,
        ],
    )


def _tc_first_body(n_real, degp, x, w1, s_out, g1_out):
    deg = degp[0] + degp[1] + 1.0
    s = lax.rsqrt(deg)
    row = lax.broadcasted_iota(jnp.int32, s.shape, 0)
    s = jnp.where(row < n_real, s, 0.0)
    s_out[...] = s
    g1_out[...] = s * jnp.dot(x[...], w1[...], preferred_element_type=jnp.float32)


def _tc_mid_body(aggp, g, s, b, w, gn_out):
    sv = s[...]
    h = jnp.maximum(sv * (aggp[0] + aggp[1] + g[...]) + b[...], 0.0)
    gn_out[...] = sv * jnp.dot(h, w[...], preferred_element_type=jnp.float32)


def _tc_last_body(num_graphs, aggp, g, s, b, batch, wlin, blin, out):
    h = jnp.maximum(s[...] * (aggp[0] + aggp[1] + g[...]) + b[...], 0.0)
    cls = lax.broadcasted_iota(jnp.int32, (h.shape[0], num_graphs), 1)
    p = (batch[...] == cls).astype(jnp.float32)
    pooled = lax.dot_general(p, h, (((0,), (0,)), ((), ())),
                             preferred_element_type=jnp.float32)
    out[...] = jnp.dot(pooled, wlin[...], preferred_element_type=jnp.float32) + blin[...]


def kernel(x, edge_index, batch, W1, b1, W2, b2, W3, b3, Wlin, blin):
    n, d_in = x.shape
    e = edge_index.shape[1]
    num_graphs = 64
    out_dim = Wlin.shape[1]

    npad = _pad_to(n, _NS * 8)
    epad = _pad_to(e, _NW * _CHUNK * 8)

    pad_e = jnp.full((epad - e,), n, dtype=jnp.int32)
    src2d = jnp.concatenate([edge_index[0], pad_e]).reshape(epad // _CHUNK, _CHUNK)
    dst2d = jnp.concatenate([edge_index[1], pad_e]).reshape(epad // _CHUNK, _CHUNK)

    x_p = jnp.pad(x, ((0, npad - n), (0, 0)))
    batch_p = jnp.pad(batch, (0, npad - n), constant_values=num_graphs)[:, None]

    ones_col = jnp.ones((_CHUNK, 1), jnp.float32)
    zeros_deg = jnp.zeros((npad // _NS, 1), jnp.float32)
    zeros16 = jnp.zeros((npad // _NS, 16), jnp.float32)
    zeros32 = jnp.zeros((npad // _NS, 32), jnp.float32)

    degp = _make_sc_deg(epad, npad)(dst2d, ones_col, zeros_deg)

    s, g1 = pl.pallas_call(
        functools.partial(_tc_first_body, n),
        out_shape=[jax.ShapeDtypeStruct((npad, 1), jnp.float32),
                   jax.ShapeDtypeStruct((npad, 16), jnp.float32)],
    )(degp, x_p, W1)

    agg1 = _make_sc_agg(epad, npad, 16)(src2d, dst2d, g1, zeros16)

    g2 = pl.pallas_call(
        _tc_mid_body,
        out_shape=jax.ShapeDtypeStruct((npad, 32), jnp.float32),
    )(agg1, g1, s, b1[None, :], W2)

    agg2 = _make_sc_agg(epad, npad, 32)(src2d, dst2d, g2, zeros32)

    g3 = pl.pallas_call(
        _tc_mid_body,
        out_shape=jax.ShapeDtypeStruct((npad, 32), jnp.float32),
    )(agg2, g2, s, b2[None, :], W3)

    agg3 = _make_sc_agg(epad, npad, 32)(src2d, dst2d, g3, zeros32)

    out = pl.pallas_call(
        functools.partial(_tc_last_body, num_graphs),
        out_shape=jax.ShapeDtypeStruct((num_graphs, out_dim), jnp.float32),
    )(agg3, g3, s, b3[None, :], batch_p, Wlin, blin[None, :])

    return out


# speedup vs baseline: 1.1253x; 1.1253x over previous
import functools

import jax
import jax.numpy as jnp
from jax import lax
from jax.experimental import pallas as pl
from jax.experimental.pallas import tpu as pltpu
from jax.experimental.pallas import tpu_sc as plsc

_NC = 2
_NS = 16
_NW = _NC * _NS
_CHUNK = 128
_NBUF = 4
_X0_FRAC = 0.65


def _pad_to(n, m):
    return (n + m - 1) // m * m


def _sc_agg_body(x0, y1, npad, feat, src_hbm, dst_hbm, g_hbm, zeros_hbm,
                 out_hbm, sidx_v, didx_v, rows_v, acc_sh, *sems):
    cid = lax.axis_index("c")
    sid = lax.axis_index("s")
    rows_per_tile = npad // _NS

    pltpu.sync_copy(zeros_hbm, acc_sh.at[pl.ds(sid * rows_per_tile, rows_per_tile), :])
    plsc.subcore_barrier()

    def run(nch, base):
        pltpu.sync_copy(src_hbm.at[pl.ds(base, nch), :], sidx_v.at[pl.ds(0, nch), :])
        pltpu.sync_copy(dst_hbm.at[pl.ds(base, nch), :], didx_v.at[pl.ds(0, nch), :])

        def group(gi, carry):
            descs = []
            for b in range(_NBUF):
                j = gi * _NBUF + b
                descs.append(pltpu.async_copy(g_hbm.at[sidx_v.at[j]], rows_v.at[b], sems[b]))
            for b in range(_NBUF):
                j = gi * _NBUF + b
                descs[b].wait()
                pltpu.sync_copy(rows_v.at[b], acc_sh.at[didx_v.at[j]], add=True)
            return carry

        lax.fori_loop(0, nch // _NBUF, group, 0)

    pl.when(cid == 0)(lambda: run(x0, sid * x0))
    pl.when(cid == 1)(lambda: run(y1, _NS * x0 + sid * y1))
    plsc.subcore_barrier()

    pltpu.sync_copy(
        acc_sh.at[pl.ds(sid * rows_per_tile, rows_per_tile), :],
        out_hbm.at[cid, pl.ds(sid * rows_per_tile, rows_per_tile), :])


def _make_sc_agg(epad, npad, feat):
    per_pair = epad // (_NS * _CHUNK)
    x0 = round(per_pair * _X0_FRAC / 8) * 8
    y1 = per_pair - x0
    mesh = plsc.VectorSubcoreMesh(core_axis_name="c", subcore_axis_name="s")
    body = functools.partial(_sc_agg_body, x0, y1, npad, feat)
    mx = max(x0, y1)
    return pl.kernel(
        body,
        out_type=jax.ShapeDtypeStruct((_NC, npad, feat), jnp.float32),
        mesh=mesh,
        compiler_params=pltpu.CompilerParams(use_tc_tiling_on_sc=False),
        scratch_types=[
            pltpu.VMEM((mx, _CHUNK), jnp.int32),
            pltpu.VMEM((mx, _CHUNK), jnp.int32),
            pltpu.VMEM((_NBUF, _CHUNK, feat), jnp.float32),
            pltpu.VMEM_SHARED((npad, feat), jnp.float32),
        ] + [pltpu.SemaphoreType.DMA] * _NBUF,
    )


def _sc_deg_body(nchunks, npad, dst_hbm, ones_hbm, zeros_hbm, out_hbm,
                 didx_v, ones_v, acc_sh, sem):
    cid = lax.axis_index("c")
    sid = lax.axis_index("s")
    wid = sid * _NC + cid
    rows_per_tile = npad // _NS

    pltpu.sync_copy(dst_hbm.at[pl.ds(wid * nchunks, nchunks), :], didx_v)
    pltpu.sync_copy(ones_hbm, ones_v)
    pltpu.sync_copy(zeros_hbm, acc_sh.at[pl.ds(sid * rows_per_tile, rows_per_tile), :])
    plsc.subcore_barrier()

    def body(j, carry):
        pltpu.sync_copy(ones_v, acc_sh.at[didx_v.at[j]], add=True)
        return carry

    lax.fori_loop(0, nchunks, body, 0)
    plsc.subcore_barrier()

    pltpu.sync_copy(
        acc_sh.at[pl.ds(sid * rows_per_tile, rows_per_tile), :],
        out_hbm.at[cid, pl.ds(sid * rows_per_tile, rows_per_tile), :])


def _make_sc_deg(epad, npad):
    nchunks = epad // (_NW * _CHUNK)
    mesh = plsc.VectorSubcoreMesh(core_axis_name="c", subcore_axis_name="s")
    body = functools.partial(_sc_deg_body, nchunks, npad)
    return pl.kernel(
        body,
        out_type=jax.ShapeDtypeStruct((_NC, npad, 1), jnp.float32),
        mesh=mesh,
        compiler_params=pltpu.CompilerParams(use_tc_tiling_on_sc=False),
        scratch_types=[
            pltpu.VMEM((nchunks, _CHUNK), jnp.int32),
            pltpu.VMEM((_CHUNK, 1), jnp.float32),
            pltpu.VMEM_SHARED((npad, 1), jnp.float32),
            pltpu.SemaphoreType.DMA,
        ],
    )


def _tc_first_body(n_real, degp, x, w1, s_out, g1_out):
    deg = degp[0] + degp[1] + 1.0
    s = lax.rsqrt(deg)
    row = lax.broadcasted_iota(jnp.int32, s.shape, 0)
    s = jnp.where(row < n_real, s, 0.0)
    s_out[...] = s
    g1_out[...] = s * jnp.dot(x[...], w1[...], preferred_element_type=jnp.float32)


def _tc_mid_body(aggp, g, s, b, w, gn_out):
    sv = s[...]
    h = jnp.maximum(sv * (aggp[0] + aggp[1] + g[...]) + b[...], 0.0)
    gn_out[...] = sv * jnp.dot(h, w[...], preferred_element_type=jnp.float32)


def _tc_last_body(num_graphs, aggp, g, s, b, batch, wlin, blin, out):
    h = jnp.maximum(s[...] * (aggp[0] + aggp[1] + g[...]) + b[...], 0.0)
    cls = lax.broadcasted_iota(jnp.int32, (h.shape[0], num_graphs), 1)
    p = (batch[...] == cls).astype(jnp.float32)
    pooled = lax.dot_general(p, h, (((0,), (0,)), ((), ())),
                             preferred_element_type=jnp.float32)
    out[...] = jnp.dot(pooled, wlin[...], preferred_element_type=jnp.float32) + blin[...]


def kernel(x, edge_index, batch, W1, b1, W2, b2, W3, b3, Wlin, blin):
    n, d_in = x.shape
    e = edge_index.shape[1]
    num_graphs = 64
    out_dim = Wlin.shape[1]

    npad = _pad_to(n, _NS * 8)
    epad = _pad_to(e, _NW * _CHUNK * 8)

    pad_e = jnp.full((epad - e,), n, dtype=jnp.int32)
    src2d = jnp.concatenate([edge_index[0], pad_e]).reshape(epad // _CHUNK, _CHUNK)
    dst2d = jnp.concatenate([edge_index[1], pad_e]).reshape(epad // _CHUNK, _CHUNK)

    x_p = jnp.pad(x, ((0, npad - n), (0, 0)))
    batch_p = jnp.pad(batch, (0, npad - n), constant_values=num_graphs)[:, None]

    ones_col = jnp.ones((_CHUNK, 1), jnp.float32)
    zeros_deg = jnp.zeros((npad // _NS, 1), jnp.float32)
    zeros16 = jnp.zeros((npad // _NS, 16), jnp.float32)
    zeros32 = jnp.zeros((npad // _NS, 32), jnp.float32)

    degp = _make_sc_deg(epad, npad)(dst2d, ones_col, zeros_deg)

    s, g1 = pl.pallas_call(
        functools.partial(_tc_first_body, n),
        out_shape=[jax.ShapeDtypeStruct((npad, 1), jnp.float32),
                   jax.ShapeDtypeStruct((npad, 16), jnp.float32)],
    )(degp, x_p, W1)

    agg1 = _make_sc_agg(epad, npad, 16)(src2d, dst2d, g1, zeros16)

    g2 = pl.pallas_call(
        _tc_mid_body,
        out_shape=jax.ShapeDtypeStruct((npad, 32), jnp.float32),
    )(agg1, g1, s, b1[None, :], W2)

    agg2 = _make_sc_agg(epad, npad, 32)(src2d, dst2d, g2, zeros32)

    g3 = pl.pallas_call(
        _tc_mid_body,
        out_shape=jax.ShapeDtypeStruct((npad, 32), jnp.float32),
    )(agg2, g2, s, b2[None, :], W3)

    agg3 = _make_sc_agg(epad, npad, 32)(src2d, dst2d, g3, zeros32)

    out = pl.pallas_call(
        functools.partial(_tc_last_body, num_graphs),
        out_shape=jax.ShapeDtypeStruct((num_graphs, out_dim), jnp.float32),
    )(agg3, g3, s, b3[None, :], batch_p, Wlin, blin[None, :])

    return out


# speedup vs baseline: 1.1576x; 1.0287x over previous
import functools

import jax
import jax.numpy as jnp
from jax import lax
from jax.experimental import pallas as pl
from jax.experimental.pallas import tpu as pltpu
from jax.experimental.pallas import tpu_sc as plsc

_NC = 2
_NS = 16
_NW = _NC * _NS
_CHUNK = 128
_NBUF = 4
_X0_FRAC = 0.72


def _pad_to(n, m):
    return (n + m - 1) // m * m


def _sc_agg_body(x0, y1, npad, feat, src_hbm, dst_hbm, g_hbm, zeros_hbm,
                 out_hbm, sidx_v, didx_v, rows_v, acc_sh, *sems):
    cid = lax.axis_index("c")
    sid = lax.axis_index("s")
    rows_per_tile = npad // _NS

    pltpu.sync_copy(zeros_hbm, acc_sh.at[pl.ds(sid * rows_per_tile, rows_per_tile), :])
    plsc.subcore_barrier()

    def run(nch, base):
        pltpu.sync_copy(src_hbm.at[pl.ds(base, nch), :], sidx_v.at[pl.ds(0, nch), :])
        pltpu.sync_copy(dst_hbm.at[pl.ds(base, nch), :], didx_v.at[pl.ds(0, nch), :])

        def group(gi, carry):
            descs = []
            for b in range(_NBUF):
                j = gi * _NBUF + b
                descs.append(pltpu.async_copy(g_hbm.at[sidx_v.at[j]], rows_v.at[b], sems[b]))
            for b in range(_NBUF):
                j = gi * _NBUF + b
                descs[b].wait()
                pltpu.sync_copy(rows_v.at[b], acc_sh.at[didx_v.at[j]], add=True)
            return carry

        lax.fori_loop(0, nch // _NBUF, group, 0)

    pl.when(cid == 0)(lambda: run(x0, sid * x0))
    pl.when(cid == 1)(lambda: run(y1, _NS * x0 + sid * y1))
    plsc.subcore_barrier()

    pltpu.sync_copy(
        acc_sh.at[pl.ds(sid * rows_per_tile, rows_per_tile), :],
        out_hbm.at[cid, pl.ds(sid * rows_per_tile, rows_per_tile), :])


def _make_sc_agg(epad, npad, feat):
    per_pair = epad // (_NS * _CHUNK)
    x0 = round(per_pair * _X0_FRAC / 8) * 8
    y1 = per_pair - x0
    mesh = plsc.VectorSubcoreMesh(core_axis_name="c", subcore_axis_name="s")
    body = functools.partial(_sc_agg_body, x0, y1, npad, feat)
    mx = max(x0, y1)
    return pl.kernel(
        body,
        out_type=jax.ShapeDtypeStruct((_NC, npad, feat), jnp.float32),
        mesh=mesh,
        compiler_params=pltpu.CompilerParams(use_tc_tiling_on_sc=False),
        scratch_types=[
            pltpu.VMEM((mx, _CHUNK), jnp.int32),
            pltpu.VMEM((mx, _CHUNK), jnp.int32),
            pltpu.VMEM((_NBUF, _CHUNK, feat), jnp.float32),
            pltpu.VMEM_SHARED((npad, feat), jnp.float32),
        ] + [pltpu.SemaphoreType.DMA] * _NBUF,
    )


def _sc_deg_body(nchunks, npad, dst_hbm, ones_hbm, zeros_hbm, out_hbm,
                 didx_v, ones_v, acc_sh, sem):
    cid = lax.axis_index("c")
    sid = lax.axis_index("s")
    wid = sid * _NC + cid
    rows_per_tile = npad // _NS

    pltpu.sync_copy(dst_hbm.at[pl.ds(wid * nchunks, nchunks), :], didx_v)
    pltpu.sync_copy(ones_hbm, ones_v)
    pltpu.sync_copy(zeros_hbm, acc_sh.at[pl.ds(sid * rows_per_tile, rows_per_tile), :])
    plsc.subcore_barrier()

    def body(j, carry):
        pltpu.sync_copy(ones_v, acc_sh.at[didx_v.at[j]], add=True)
        return carry

    lax.fori_loop(0, nchunks, body, 0)
    plsc.subcore_barrier()

    pltpu.sync_copy(
        acc_sh.at[pl.ds(sid * rows_per_tile, rows_per_tile), :],
        out_hbm.at[cid, pl.ds(sid * rows_per_tile, rows_per_tile), :])


def _make_sc_deg(epad, npad):
    nchunks = epad // (_NW * _CHUNK)
    mesh = plsc.VectorSubcoreMesh(core_axis_name="c", subcore_axis_name="s")
    body = functools.partial(_sc_deg_body, nchunks, npad)
    return pl.kernel(
        body,
        out_type=jax.ShapeDtypeStruct((_NC, npad, 1), jnp.float32),
        mesh=mesh,
        compiler_params=pltpu.CompilerParams(use_tc_tiling_on_sc=False),
        scratch_types=[
            pltpu.VMEM((nchunks, _CHUNK), jnp.int32),
            pltpu.VMEM((_CHUNK, 1), jnp.float32),
            pltpu.VMEM_SHARED((npad, 1), jnp.float32),
            pltpu.SemaphoreType.DMA,
        ],
    )


def _tc_first_body(n_real, degp, x, w1, s_out, g1_out):
    deg = degp[0] + degp[1] + 1.0
    s = lax.rsqrt(deg)
    row = lax.broadcasted_iota(jnp.int32, s.shape, 0)
    s = jnp.where(row < n_real, s, 0.0)
    s_out[...] = s
    g1_out[...] = s * jnp.dot(x[...], w1[...], preferred_element_type=jnp.float32)


def _tc_mid_body(aggp, g, s, b, w, gn_out):
    sv = s[...]
    h = jnp.maximum(sv * (aggp[0] + aggp[1] + g[...]) + b[...], 0.0)
    gn_out[...] = sv * jnp.dot(h, w[...], preferred_element_type=jnp.float32)


def _tc_last_body(num_graphs, aggp, g, s, b, batch, wlin, blin, out):
    h = jnp.maximum(s[...] * (aggp[0] + aggp[1] + g[...]) + b[...], 0.0)
    cls = lax.broadcasted_iota(jnp.int32, (h.shape[0], num_graphs), 1)
    p = (batch[...] == cls).astype(jnp.float32)
    pooled = lax.dot_general(p, h, (((0,), (0,)), ((), ())),
                             preferred_element_type=jnp.float32)
    out[...] = jnp.dot(pooled, wlin[...], preferred_element_type=jnp.float32) + blin[...]


def kernel(x, edge_index, batch, W1, b1, W2, b2, W3, b3, Wlin, blin):
    n, d_in = x.shape
    e = edge_index.shape[1]
    num_graphs = 64
    out_dim = Wlin.shape[1]

    npad = _pad_to(n, _NS * 8)
    epad = _pad_to(e, _NW * _CHUNK * 8)

    pad_e = jnp.full((epad - e,), n, dtype=jnp.int32)
    src2d = jnp.concatenate([edge_index[0], pad_e]).reshape(epad // _CHUNK, _CHUNK)
    dst2d = jnp.concatenate([edge_index[1], pad_e]).reshape(epad // _CHUNK, _CHUNK)

    x_p = jnp.pad(x, ((0, npad - n), (0, 0)))
    batch_p = jnp.pad(batch, (0, npad - n), constant_values=num_graphs)[:, None]

    ones_col = jnp.ones((_CHUNK, 1), jnp.float32)
    zeros_deg = jnp.zeros((npad // _NS, 1), jnp.float32)
    zeros16 = jnp.zeros((npad // _NS, 16), jnp.float32)
    zeros32 = jnp.zeros((npad // _NS, 32), jnp.float32)

    degp = _make_sc_deg(epad, npad)(dst2d, ones_col, zeros_deg)

    s, g1 = pl.pallas_call(
        functools.partial(_tc_first_body, n),
        out_shape=[jax.ShapeDtypeStruct((npad, 1), jnp.float32),
                   jax.ShapeDtypeStruct((npad, 16), jnp.float32)],
    )(degp, x_p, W1)

    agg1 = _make_sc_agg(epad, npad, 16)(src2d, dst2d, g1, zeros16)

    g2 = pl.pallas_call(
        _tc_mid_body,
        out_shape=jax.ShapeDtypeStruct((npad, 32), jnp.float32),
    )(agg1, g1, s, b1[None, :], W2)

    agg2 = _make_sc_agg(epad, npad, 32)(src2d, dst2d, g2, zeros32)

    g3 = pl.pallas_call(
        _tc_mid_body,
        out_shape=jax.ShapeDtypeStruct((npad, 32), jnp.float32),
    )(agg2, g2, s, b2[None, :], W3)

    agg3 = _make_sc_agg(epad, npad, 32)(src2d, dst2d, g3, zeros32)

    out = pl.pallas_call(
        functools.partial(_tc_last_body, num_graphs),
        out_shape=jax.ShapeDtypeStruct((num_graphs, out_dim), jnp.float32),
    )(agg3, g3, s, b3[None, :], batch_p, Wlin, blin[None, :])

    return out


# speedup vs baseline: 1.2183x; 1.0525x over previous
import functools

import jax
import jax.numpy as jnp
from jax import lax
from jax.experimental import pallas as pl
from jax.experimental.pallas import tpu as pltpu
from jax.experimental.pallas import tpu_sc as plsc

_NC = 2
_NS = 16
_NW = _NC * _NS
_CHUNK = 128
_NBUF = 4
_X0_FRAC = 0.80


def _pad_to(n, m):
    return (n + m - 1) // m * m


def _sc_agg_body(x0, y1, npad, feat, src_hbm, dst_hbm, g_hbm, zeros_hbm,
                 out_hbm, sidx_v, didx_v, rows_v, acc_sh, *sems):
    cid = lax.axis_index("c")
    sid = lax.axis_index("s")
    rows_per_tile = npad // _NS

    pltpu.sync_copy(zeros_hbm, acc_sh.at[pl.ds(sid * rows_per_tile, rows_per_tile), :])
    plsc.subcore_barrier()

    def run(nch, base):
        pltpu.sync_copy(src_hbm.at[pl.ds(base, nch), :], sidx_v.at[pl.ds(0, nch), :])
        pltpu.sync_copy(dst_hbm.at[pl.ds(base, nch), :], didx_v.at[pl.ds(0, nch), :])

        def group(gi, carry):
            descs = []
            for b in range(_NBUF):
                j = gi * _NBUF + b
                descs.append(pltpu.async_copy(g_hbm.at[sidx_v.at[j]], rows_v.at[b], sems[b]))
            for b in range(_NBUF):
                j = gi * _NBUF + b
                descs[b].wait()
                pltpu.sync_copy(rows_v.at[b], acc_sh.at[didx_v.at[j]], add=True)
            return carry

        lax.fori_loop(0, nch // _NBUF, group, 0)

    pl.when(cid == 0)(lambda: run(x0, sid * x0))
    pl.when(cid == 1)(lambda: run(y1, _NS * x0 + sid * y1))
    plsc.subcore_barrier()

    pltpu.sync_copy(
        acc_sh.at[pl.ds(sid * rows_per_tile, rows_per_tile), :],
        out_hbm.at[cid, pl.ds(sid * rows_per_tile, rows_per_tile), :])


def _make_sc_agg(epad, npad, feat):
    per_pair = epad // (_NS * _CHUNK)
    x0 = round(per_pair * _X0_FRAC / 8) * 8
    y1 = per_pair - x0
    mesh = plsc.VectorSubcoreMesh(core_axis_name="c", subcore_axis_name="s")
    body = functools.partial(_sc_agg_body, x0, y1, npad, feat)
    mx = max(x0, y1)
    return pl.kernel(
        body,
        out_type=jax.ShapeDtypeStruct((_NC, npad, feat), jnp.float32),
        mesh=mesh,
        compiler_params=pltpu.CompilerParams(use_tc_tiling_on_sc=False),
        scratch_types=[
            pltpu.VMEM((mx, _CHUNK), jnp.int32),
            pltpu.VMEM((mx, _CHUNK), jnp.int32),
            pltpu.VMEM((_NBUF, _CHUNK, feat), jnp.float32),
            pltpu.VMEM_SHARED((npad, feat), jnp.float32),
        ] + [pltpu.SemaphoreType.DMA] * _NBUF,
    )


def _sc_deg_body(nchunks, npad, dst_hbm, ones_hbm, zeros_hbm, out_hbm,
                 didx_v, ones_v, acc_sh, sem):
    cid = lax.axis_index("c")
    sid = lax.axis_index("s")
    wid = sid * _NC + cid
    rows_per_tile = npad // _NS

    pltpu.sync_copy(dst_hbm.at[pl.ds(wid * nchunks, nchunks), :], didx_v)
    pltpu.sync_copy(ones_hbm, ones_v)
    pltpu.sync_copy(zeros_hbm, acc_sh.at[pl.ds(sid * rows_per_tile, rows_per_tile), :])
    plsc.subcore_barrier()

    def body(j, carry):
        pltpu.sync_copy(ones_v, acc_sh.at[didx_v.at[j]], add=True)
        return carry

    lax.fori_loop(0, nchunks, body, 0)
    plsc.subcore_barrier()

    pltpu.sync_copy(
        acc_sh.at[pl.ds(sid * rows_per_tile, rows_per_tile), :],
        out_hbm.at[cid, pl.ds(sid * rows_per_tile, rows_per_tile), :])


def _make_sc_deg(epad, npad):
    nchunks = epad // (_NW * _CHUNK)
    mesh = plsc.VectorSubcoreMesh(core_axis_name="c", subcore_axis_name="s")
    body = functools.partial(_sc_deg_body, nchunks, npad)
    return pl.kernel(
        body,
        out_type=jax.ShapeDtypeStruct((_NC, npad, 1), jnp.float32),
        mesh=mesh,
        compiler_params=pltpu.CompilerParams(use_tc_tiling_on_sc=False),
        scratch_types=[
            pltpu.VMEM((nchunks, _CHUNK), jnp.int32),
            pltpu.VMEM((_CHUNK, 1), jnp.float32),
            pltpu.VMEM_SHARED((npad, 1), jnp.float32),
            pltpu.SemaphoreType.DMA,
        ],
    )


def _tc_first_body(n_real, degp, x, w1, s_out, g1_out):
    deg = degp[0] + degp[1] + 1.0
    s = lax.rsqrt(deg)
    row = lax.broadcasted_iota(jnp.int32, s.shape, 0)
    s = jnp.where(row < n_real, s, 0.0)
    s_out[...] = s
    g1_out[...] = s * jnp.dot(x[...], w1[...], preferred_element_type=jnp.float32)


def _tc_mid_body(aggp, g, s, b, w, gn_out):
    sv = s[...]
    h = jnp.maximum(sv * (aggp[0] + aggp[1] + g[...]) + b[...], 0.0)
    gn_out[...] = sv * jnp.dot(h, w[...], preferred_element_type=jnp.float32)


def _tc_last_body(num_graphs, aggp, g, s, b, batch, wlin, blin, out):
    h = jnp.maximum(s[...] * (aggp[0] + aggp[1] + g[...]) + b[...], 0.0)
    cls = lax.broadcasted_iota(jnp.int32, (h.shape[0], num_graphs), 1)
    p = (batch[...] == cls).astype(jnp.float32)
    pooled = lax.dot_general(p, h, (((0,), (0,)), ((), ())),
                             preferred_element_type=jnp.float32)
    out[...] = jnp.dot(pooled, wlin[...], preferred_element_type=jnp.float32) + blin[...]


def kernel(x, edge_index, batch, W1, b1, W2, b2, W3, b3, Wlin, blin):
    n, d_in = x.shape
    e = edge_index.shape[1]
    num_graphs = 64
    out_dim = Wlin.shape[1]

    npad = _pad_to(n, _NS * 8)
    epad = _pad_to(e, _NW * _CHUNK * 8)

    pad_e = jnp.full((epad - e,), n, dtype=jnp.int32)
    src2d = jnp.concatenate([edge_index[0], pad_e]).reshape(epad // _CHUNK, _CHUNK)
    dst2d = jnp.concatenate([edge_index[1], pad_e]).reshape(epad // _CHUNK, _CHUNK)

    x_p = jnp.pad(x, ((0, npad - n), (0, 0)))
    batch_p = jnp.pad(batch, (0, npad - n), constant_values=num_graphs)[:, None]

    ones_col = jnp.ones((_CHUNK, 1), jnp.float32)
    zeros_deg = jnp.zeros((npad // _NS, 1), jnp.float32)
    zeros16 = jnp.zeros((npad // _NS, 16), jnp.float32)
    zeros32 = jnp.zeros((npad // _NS, 32), jnp.float32)

    degp = _make_sc_deg(epad, npad)(dst2d, ones_col, zeros_deg)

    s, g1 = pl.pallas_call(
        functools.partial(_tc_first_body, n),
        out_shape=[jax.ShapeDtypeStruct((npad, 1), jnp.float32),
                   jax.ShapeDtypeStruct((npad, 16), jnp.float32)],
    )(degp, x_p, W1)

    agg1 = _make_sc_agg(epad, npad, 16)(src2d, dst2d, g1, zeros16)

    g2 = pl.pallas_call(
        _tc_mid_body,
        out_shape=jax.ShapeDtypeStruct((npad, 32), jnp.float32),
    )(agg1, g1, s, b1[None, :], W2)

    agg2 = _make_sc_agg(epad, npad, 32)(src2d, dst2d, g2, zeros32)

    g3 = pl.pallas_call(
        _tc_mid_body,
        out_shape=jax.ShapeDtypeStruct((npad, 32), jnp.float32),
    )(agg2, g2, s, b2[None, :], W3)

    agg3 = _make_sc_agg(epad, npad, 32)(src2d, dst2d, g3, zeros32)

    out = pl.pallas_call(
        functools.partial(_tc_last_body, num_graphs),
        out_shape=jax.ShapeDtypeStruct((num_graphs, out_dim), jnp.float32),
    )(agg3, g3, s, b3[None, :], batch_p, Wlin, blin[None, :])

    return out


# speedup vs baseline: 1.2630x; 1.0367x over previous
import functools

import jax
import jax.numpy as jnp
from jax import lax
from jax.experimental import pallas as pl
from jax.experimental.pallas import tpu as pltpu
from jax.experimental.pallas import tpu_sc as plsc

_NC = 2
_NS = 16
_NW = _NC * _NS
_CHUNK = 128
_NBUF = 4
_X0_FRAC = 0.90


def _pad_to(n, m):
    return (n + m - 1) // m * m


def _sc_agg_body(x0, y1, npad, feat, src_hbm, dst_hbm, g_hbm, zeros_hbm,
                 out_hbm, sidx_v, didx_v, rows_v, acc_sh, *sems):
    cid = lax.axis_index("c")
    sid = lax.axis_index("s")
    rows_per_tile = npad // _NS

    pltpu.sync_copy(zeros_hbm, acc_sh.at[pl.ds(sid * rows_per_tile, rows_per_tile), :])
    plsc.subcore_barrier()

    def run(nch, base):
        pltpu.sync_copy(src_hbm.at[pl.ds(base, nch), :], sidx_v.at[pl.ds(0, nch), :])
        pltpu.sync_copy(dst_hbm.at[pl.ds(base, nch), :], didx_v.at[pl.ds(0, nch), :])

        def group(gi, carry):
            descs = []
            for b in range(_NBUF):
                j = gi * _NBUF + b
                descs.append(pltpu.async_copy(g_hbm.at[sidx_v.at[j]], rows_v.at[b], sems[b]))
            for b in range(_NBUF):
                j = gi * _NBUF + b
                descs[b].wait()
                pltpu.sync_copy(rows_v.at[b], acc_sh.at[didx_v.at[j]], add=True)
            return carry

        lax.fori_loop(0, nch // _NBUF, group, 0)

    pl.when(cid == 0)(lambda: run(x0, sid * x0))
    pl.when(cid == 1)(lambda: run(y1, _NS * x0 + sid * y1))
    plsc.subcore_barrier()

    pltpu.sync_copy(
        acc_sh.at[pl.ds(sid * rows_per_tile, rows_per_tile), :],
        out_hbm.at[cid, pl.ds(sid * rows_per_tile, rows_per_tile), :])


def _make_sc_agg(epad, npad, feat):
    per_pair = epad // (_NS * _CHUNK)
    x0 = round(per_pair * _X0_FRAC / 8) * 8
    y1 = per_pair - x0
    mesh = plsc.VectorSubcoreMesh(core_axis_name="c", subcore_axis_name="s")
    body = functools.partial(_sc_agg_body, x0, y1, npad, feat)
    mx = max(x0, y1)
    return pl.kernel(
        body,
        out_type=jax.ShapeDtypeStruct((_NC, npad, feat), jnp.float32),
        mesh=mesh,
        compiler_params=pltpu.CompilerParams(use_tc_tiling_on_sc=False),
        scratch_types=[
            pltpu.VMEM((mx, _CHUNK), jnp.int32),
            pltpu.VMEM((mx, _CHUNK), jnp.int32),
            pltpu.VMEM((_NBUF, _CHUNK, feat), jnp.float32),
            pltpu.VMEM_SHARED((npad, feat), jnp.float32),
        ] + [pltpu.SemaphoreType.DMA] * _NBUF,
    )


def _sc_deg_body(nchunks, npad, dst_hbm, ones_hbm, zeros_hbm, out_hbm,
                 didx_v, ones_v, acc_sh, sem):
    cid = lax.axis_index("c")
    sid = lax.axis_index("s")
    wid = sid * _NC + cid
    rows_per_tile = npad // _NS

    pltpu.sync_copy(dst_hbm.at[pl.ds(wid * nchunks, nchunks), :], didx_v)
    pltpu.sync_copy(ones_hbm, ones_v)
    pltpu.sync_copy(zeros_hbm, acc_sh.at[pl.ds(sid * rows_per_tile, rows_per_tile), :])
    plsc.subcore_barrier()

    def body(j, carry):
        pltpu.sync_copy(ones_v, acc_sh.at[didx_v.at[j]], add=True)
        return carry

    lax.fori_loop(0, nchunks, body, 0)
    plsc.subcore_barrier()

    pltpu.sync_copy(
        acc_sh.at[pl.ds(sid * rows_per_tile, rows_per_tile), :],
        out_hbm.at[cid, pl.ds(sid * rows_per_tile, rows_per_tile), :])


def _make_sc_deg(epad, npad):
    nchunks = epad // (_NW * _CHUNK)
    mesh = plsc.VectorSubcoreMesh(core_axis_name="c", subcore_axis_name="s")
    body = functools.partial(_sc_deg_body, nchunks, npad)
    return pl.kernel(
        body,
        out_type=jax.ShapeDtypeStruct((_NC, npad, 1), jnp.float32),
        mesh=mesh,
        compiler_params=pltpu.CompilerParams(use_tc_tiling_on_sc=False),
        scratch_types=[
            pltpu.VMEM((nchunks, _CHUNK), jnp.int32),
            pltpu.VMEM((_CHUNK, 1), jnp.float32),
            pltpu.VMEM_SHARED((npad, 1), jnp.float32),
            pltpu.SemaphoreType.DMA,
        ],
    )


def _tc_first_body(n_real, degp, x, w1, s_out, g1_out):
    deg = degp[0] + degp[1] + 1.0
    s = lax.rsqrt(deg)
    row = lax.broadcasted_iota(jnp.int32, s.shape, 0)
    s = jnp.where(row < n_real, s, 0.0)
    s_out[...] = s
    g1_out[...] = s * jnp.dot(x[...], w1[...], preferred_element_type=jnp.float32)


def _tc_mid_body(aggp, g, s, b, w, gn_out):
    sv = s[...]
    h = jnp.maximum(sv * (aggp[0] + aggp[1] + g[...]) + b[...], 0.0)
    gn_out[...] = sv * jnp.dot(h, w[...], preferred_element_type=jnp.float32)


def _tc_last_body(num_graphs, aggp, g, s, b, batch, wlin, blin, out):
    h = jnp.maximum(s[...] * (aggp[0] + aggp[1] + g[...]) + b[...], 0.0)
    cls = lax.broadcasted_iota(jnp.int32, (h.shape[0], num_graphs), 1)
    p = (batch[...] == cls).astype(jnp.float32)
    pooled = lax.dot_general(p, h, (((0,), (0,)), ((), ())),
                             preferred_element_type=jnp.float32)
    out[...] = jnp.dot(pooled, wlin[...], preferred_element_type=jnp.float32) + blin[...]


def kernel(x, edge_index, batch, W1, b1, W2, b2, W3, b3, Wlin, blin):
    n, d_in = x.shape
    e = edge_index.shape[1]
    num_graphs = 64
    out_dim = Wlin.shape[1]

    npad = _pad_to(n, _NS * 8)
    epad = _pad_to(e, _NW * _CHUNK * 8)

    pad_e = jnp.full((epad - e,), n, dtype=jnp.int32)
    src2d = jnp.concatenate([edge_index[0], pad_e]).reshape(epad // _CHUNK, _CHUNK)
    dst2d = jnp.concatenate([edge_index[1], pad_e]).reshape(epad // _CHUNK, _CHUNK)

    x_p = jnp.pad(x, ((0, npad - n), (0, 0)))
    batch_p = jnp.pad(batch, (0, npad - n), constant_values=num_graphs)[:, None]

    ones_col = jnp.ones((_CHUNK, 1), jnp.float32)
    zeros_deg = jnp.zeros((npad // _NS, 1), jnp.float32)
    zeros16 = jnp.zeros((npad // _NS, 16), jnp.float32)
    zeros32 = jnp.zeros((npad // _NS, 32), jnp.float32)

    degp = _make_sc_deg(epad, npad)(dst2d, ones_col, zeros_deg)

    s, g1 = pl.pallas_call(
        functools.partial(_tc_first_body, n),
        out_shape=[jax.ShapeDtypeStruct((npad, 1), jnp.float32),
                   jax.ShapeDtypeStruct((npad, 16), jnp.float32)],
    )(degp, x_p, W1)

    agg1 = _make_sc_agg(epad, npad, 16)(src2d, dst2d, g1, zeros16)

    g2 = pl.pallas_call(
        _tc_mid_body,
        out_shape=jax.ShapeDtypeStruct((npad, 32), jnp.float32),
    )(agg1, g1, s, b1[None, :], W2)

    agg2 = _make_sc_agg(epad, npad, 32)(src2d, dst2d, g2, zeros32)

    g3 = pl.pallas_call(
        _tc_mid_body,
        out_shape=jax.ShapeDtypeStruct((npad, 32), jnp.float32),
    )(agg2, g2, s, b2[None, :], W3)

    agg3 = _make_sc_agg(epad, npad, 32)(src2d, dst2d, g3, zeros32)

    out = pl.pallas_call(
        functools.partial(_tc_last_body, num_graphs),
        out_shape=jax.ShapeDtypeStruct((num_graphs, out_dim), jnp.float32),
    )(agg3, g3, s, b3[None, :], batch_p, Wlin, blin[None, :])

    return out
